# baseline (speedup 1.0000x reference)
import jax
import jax.numpy as jnp
from jax import lax
from jax.experimental import pallas as pl
from jax.experimental.pallas import tpu as pltpu
from jax.experimental.pallas import tpu_sc as plsc

BATCH = 16384
F = 64
LANES = 16
CHUNK = 128


def _gather_body(nw, nc, idx_hbm, tab_hbm, out_hbm, idx_v, rows_v, sem0, sem1):
    b_per_w = BATCH // nw
    nchunk = b_per_w // CHUNK
    wid = lax.axis_index("s") * nc + lax.axis_index("c")
    base = wid * b_per_w
    sems = (sem0, sem1)

    pltpu.sync_copy(idx_hbm.at[pl.ds(wid * nchunk, nchunk)], idx_v)
    copies = []
    for c in range(nchunk):
        copies.append(pltpu.async_copy(
            tab_hbm.at[idx_v.at[c]], rows_v.at[pl.ds(c * CHUNK, CHUNK)],
            sems[c % 2]))
    for cpy in copies:
        cpy.wait()
    pltpu.sync_copy(rows_v, out_hbm.at[pl.ds(base, b_per_w)])


def _combine_body(nw, nc, idx_hbm, tab_hbm, eu_hbm, wb_hbm, out_hbm,
                  idx_v, eu_s, ei_v, out_v, wb_v, sem0, sem1):
    b_per_w = BATCH // nw
    nchunk = b_per_w // CHUNK
    wid = lax.axis_index("s") * nc + lax.axis_index("c")
    base = wid * b_per_w
    sems = (sem0, sem1)

    pltpu.sync_copy(idx_hbm.at[pl.ds(wid * nchunk, nchunk)], idx_v)
    pltpu.sync_copy(eu_hbm.at[pl.ds(base, b_per_w)], eu_s)
    pltpu.sync_copy(wb_hbm, wb_v)

    w = [wb_v[pl.ds(k * LANES, LANES)] for k in range(F // LANES)]
    bias_v = wb_v[pl.ds(F, LANES)]
    lane_iota = lax.iota(jnp.int32, LANES)
    lane_masks = [lane_iota == i for i in range(LANES)]

    def fire(c):
        pltpu.async_copy(tab_hbm.at[idx_v.at[c]], ei_v.at[c % 2], sems[c % 2])

    def drain(c):
        pltpu.make_async_copy(tab_hbm.at[pl.ds(0, CHUNK)], ei_v.at[c % 2],
                              sems[c % 2]).wait()

    fire(0)
    for c in range(nchunk):
        if c + 1 < nchunk:
            fire(c + 1)
        drain(c)
        slot = c % 2

        def group_body(g, _, slot=slot, c=c):
            outvec = jnp.zeros((LANES,), jnp.float32)
            for i in range(LANES):
                j = g * LANES + i
                acc = (eu_s[c * CHUNK + j, pl.ds(0, LANES)]
                       * ei_v[slot, j, pl.ds(0, LANES)]) * w[0]
                for k in range(1, F // LANES):
                    acc = acc + (eu_s[c * CHUNK + j, pl.ds(k * LANES, LANES)]
                                 * ei_v[slot, j, pl.ds(k * LANES, LANES)]
                                 ) * w[k]
                tot = jnp.full((LANES,), jnp.sum(acc), jnp.float32)
                outvec = jnp.where(lane_masks[i], tot, outvec)
            out_v[pl.ds(c * CHUNK + g * LANES, LANES)] = outvec + bias_v
            return _

        lax.fori_loop(0, CHUNK // LANES, group_body, None)

    pltpu.sync_copy(out_v, out_hbm.at[pl.ds(base, b_per_w)])


def kernel(user, item, user_table, item_table, W, b):
    info = plsc.get_sparse_core_info()
    nc, ns = info.num_cores, info.num_subcores
    nw = nc * ns
    b_per_w = BATCH // nw
    nchunk = b_per_w // CHUNK

    u2 = user.reshape(nw * nchunk, CHUNK).astype(jnp.int32)
    i2 = item.reshape(nw * nchunk, CHUNK).astype(jnp.int32)

    wb = jnp.concatenate([W.reshape(-1), jnp.full((LANES,), b[0], jnp.float32)])

    mesh = plsc.VectorSubcoreMesh(core_axis_name="c", subcore_axis_name="s")
    params = pltpu.CompilerParams(needs_layout_passes=False,
                                  use_tc_tiling_on_sc=False)

    def body_a(*refs):
        _gather_body(nw, nc, *refs)

    gather_u = pl.kernel(
        body_a,
        mesh=mesh,
        compiler_params=params,
        out_type=jax.ShapeDtypeStruct((BATCH, F), jnp.float32),
        scratch_types=[
            pltpu.VMEM((nchunk, CHUNK), jnp.int32),
            pltpu.VMEM((b_per_w, F), jnp.float32),
            pltpu.SemaphoreType.DMA,
            pltpu.SemaphoreType.DMA,
        ],
    )
    eu = gather_u(u2, user_table)

    def body_b(*refs):
        _combine_body(nw, nc, *refs)

    combine = pl.kernel(
        body_b,
        mesh=mesh,
        compiler_params=params,
        out_type=jax.ShapeDtypeStruct((BATCH,), jnp.float32),
        scratch_types=[
            pltpu.VMEM((nchunk, CHUNK), jnp.int32),
            pltpu.VMEM((b_per_w, F), jnp.float32),
            pltpu.VMEM((2, CHUNK, F), jnp.float32),
            pltpu.VMEM((b_per_w,), jnp.float32),
            pltpu.VMEM((F + LANES,), jnp.float32),
            pltpu.SemaphoreType.DMA,
            pltpu.SemaphoreType.DMA,
        ],
    )
    return combine(i2, item_table, eu, wb)


# speedup vs baseline: 1.5639x; 1.5639x over previous
import jax
import jax.numpy as jnp
from jax import lax
from jax.experimental import pallas as pl
from jax.experimental.pallas import tpu as pltpu
from jax.experimental.pallas import tpu_sc as plsc

BATCH = 16384
F = 64
LANES = 16


def _gmf_body(nw, nc, user_hbm, item_hbm, ut_hbm, it_hbm, wb_hbm, out_hbm,
              uidx_v, iidx_v, eu_v, ei_v, out_v, wb_v, sem0, sem1):
    b_per_w = BATCH // nw
    ngroup = b_per_w // LANES
    wid = lax.axis_index("s") * nc + lax.axis_index("c")
    base = wid * b_per_w
    sems = (sem0, sem1)

    pltpu.sync_copy(user_hbm.at[pl.ds(base, b_per_w)], uidx_v)
    pltpu.sync_copy(item_hbm.at[pl.ds(base, b_per_w)], iidx_v)
    pltpu.sync_copy(wb_hbm, wb_v)

    w = [wb_v[pl.ds(k * LANES, LANES)] for k in range(F // LANES)]
    bias_v = wb_v[pl.ds(F, LANES)]
    lane_iota = lax.iota(jnp.int32, LANES)
    lane_masks = [lane_iota == i for i in range(LANES)]

    def fire(g, slot):
        r0 = g * LANES
        iv_u = uidx_v[pl.ds(r0, LANES)]
        iv_i = iidx_v[pl.ds(r0, LANES)]
        for i in range(LANES):
            pltpu.async_copy(ut_hbm.at[pl.ds(iv_u[i], 1)],
                             eu_v.at[slot].at[pl.ds(i, 1)], sems[slot])
            pltpu.async_copy(it_hbm.at[pl.ds(iv_i[i], 1)],
                             ei_v.at[slot].at[pl.ds(i, 1)], sems[slot])

    def drain_compute(g, slot):
        pltpu.make_async_copy(ut_hbm.at[pl.ds(0, LANES)],
                              eu_v.at[slot], sems[slot]).wait()
        pltpu.make_async_copy(it_hbm.at[pl.ds(0, LANES)],
                              ei_v.at[slot], sems[slot]).wait()
        outvec = jnp.zeros((LANES,), jnp.float32)
        for i in range(LANES):
            acc = (eu_v[slot, i, pl.ds(0, LANES)]
                   * ei_v[slot, i, pl.ds(0, LANES)]) * w[0]
            for k in range(1, F // LANES):
                acc = acc + (eu_v[slot, i, pl.ds(k * LANES, LANES)]
                             * ei_v[slot, i, pl.ds(k * LANES, LANES)]) * w[k]
            tot = jnp.full((LANES,), jnp.sum(acc), jnp.float32)
            outvec = jnp.where(lane_masks[i], tot, outvec)
        out_v[pl.ds(g * LANES, LANES)] = outvec + bias_v

    fire(0, 0)

    def group_body(c, _):
        parity = lax.rem(c, 2)

        @pl.when(parity == 0)
        def _():
            @pl.when(c + 1 < ngroup)
            def _():
                fire(c + 1, 1)
            drain_compute(c, 0)

        @pl.when(parity == 1)
        def _():
            @pl.when(c + 1 < ngroup)
            def _():
                fire(c + 1, 0)
            drain_compute(c, 1)

        return _

    lax.fori_loop(0, ngroup, group_body, None)

    pltpu.sync_copy(out_v, out_hbm.at[pl.ds(base, b_per_w)])


def kernel(user, item, user_table, item_table, W, b):
    info = plsc.get_sparse_core_info()
    nc, ns = info.num_cores, info.num_subcores
    nw = nc * ns
    b_per_w = BATCH // nw

    wb = jnp.concatenate([W.reshape(-1), jnp.full((LANES,), b[0], jnp.float32)])

    mesh = plsc.VectorSubcoreMesh(core_axis_name="c", subcore_axis_name="s")

    def body(*refs):
        _gmf_body(nw, nc, *refs)

    f = pl.kernel(
        body,
        mesh=mesh,
        compiler_params=pltpu.CompilerParams(needs_layout_passes=False),
        out_type=jax.ShapeDtypeStruct((BATCH,), jnp.float32),
        scratch_types=[
            pltpu.VMEM((b_per_w,), jnp.int32),
            pltpu.VMEM((b_per_w,), jnp.int32),
            pltpu.VMEM((2, LANES, F), jnp.float32),
            pltpu.VMEM((2, LANES, F), jnp.float32),
            pltpu.VMEM((b_per_w,), jnp.float32),
            pltpu.VMEM((F + LANES,), jnp.float32),
            pltpu.SemaphoreType.DMA,
            pltpu.SemaphoreType.DMA,
        ],
    )
    return f(user.astype(jnp.int32), item.astype(jnp.int32),
             user_table, item_table, wb)
